# Initial kernel scaffold; baseline (speedup 1.0000x reference)
import jax
import jax.numpy as jnp
from jax.experimental import pallas as pl


def kernel(weights, x, weight_indices):
    raise NotImplementedError("write your pallas kernel here")


# speedup vs baseline: 2.0069x; 2.0069x over previous
import functools

import jax
import jax.numpy as jnp
from jax import lax
from jax.experimental import pallas as pl
from jax.experimental.pallas import tpu as pltpu, tpu_sc as plsc

D_IN = 32
D_OUT = 32
NUM_CORES = 2
NUM_SUBCORES = 16
NUM_WORKERS = NUM_CORES * NUM_SUBCORES
LANES = 16

CHUNK = 64


def _make_kernel(n_rows: int):
    assert n_rows % (NUM_WORKERS * CHUNK) == 0
    b_per_w = n_rows // NUM_WORKERS
    n_chunks = b_per_w // CHUNK
    mesh = plsc.VectorSubcoreMesh(
        core_axis_name="c", subcore_axis_name="s",
        num_cores=NUM_CORES, num_subcores=NUM_SUBCORES)

    @functools.partial(
        pl.kernel,
        out_type=jax.ShapeDtypeStruct((n_rows, D_OUT), jnp.float32),
        mesh=mesh,
        compiler_params=pltpu.CompilerParams(needs_layout_passes=False),
        scratch_types=[
            pltpu.VMEM((CHUNK,), jnp.int32),
            pltpu.VMEM((CHUNK, D_IN * D_OUT), jnp.float32),
            pltpu.VMEM((CHUNK * D_IN,), jnp.float32),
            pltpu.VMEM((CHUNK, D_OUT), jnp.float32),
            pltpu.SemaphoreType.DMA,
        ],
    )
    def seg_poly(w_hbm, x_hbm, idx_hbm, out_hbm, idx_v, w_v, x_v, o_v, sem):
        wid = lax.axis_index("s") * NUM_CORES + lax.axis_index("c")
        base = wid * b_per_w

        def chunk_body(k, carry):
            row0 = base + k * CHUNK
            pltpu.sync_copy(idx_hbm.at[pl.ds(row0, CHUNK)], idx_v)
            gather = pltpu.async_copy(w_hbm.at[idx_v], w_v, sem)
            pltpu.sync_copy(x_hbm.at[pl.ds(row0 * D_IN, CHUNK * D_IN)], x_v)
            gather.wait()

            def row_body(r, carry2):
                acc0 = jnp.zeros((LANES,), jnp.float32)
                acc1 = jnp.zeros((LANES,), jnp.float32)
                rbase = jnp.full((LANES,), r * D_IN, jnp.int32)
                for i in range(D_IN):
                    xi = plsc.load_gather(x_v, [rbase + i])
                    acc0 = acc0 + xi * w_v[r, pl.ds(i * D_OUT, LANES)]
                    acc1 = acc1 + xi * w_v[r, pl.ds(i * D_OUT + LANES, LANES)]
                o_v[r, pl.ds(0, LANES)] = acc0
                o_v[r, pl.ds(LANES, LANES)] = acc1
                return carry2

            lax.fori_loop(0, CHUNK, row_body, 0)
            pltpu.sync_copy(o_v, out_hbm.at[pl.ds(row0, CHUNK), :])
            return carry

        lax.fori_loop(0, n_chunks, chunk_body, 0)

    return seg_poly


@jax.jit
def kernel(weights, x, weight_indices):
    n_rows = x.shape[0]
    return _make_kernel(n_rows)(weights, x.reshape(-1), weight_indices)


# speedup vs baseline: 3.3139x; 1.6512x over previous
import functools

import jax
import jax.numpy as jnp
from jax import lax
from jax.experimental import pallas as pl
from jax.experimental.pallas import tpu as pltpu, tpu_sc as plsc

D_IN = 32
D_OUT = 32
NUM_CORES = 2
NUM_SUBCORES = 16
NUM_WORKERS = NUM_CORES * NUM_SUBCORES
LANES = 16

CHUNK = 32
N_BUF = 2


def _make_kernel(n_rows: int):
    assert n_rows % (NUM_WORKERS * CHUNK * N_BUF) == 0
    b_per_w = n_rows // NUM_WORKERS
    n_chunks = b_per_w // CHUNK
    mesh = plsc.VectorSubcoreMesh(
        core_axis_name="c", subcore_axis_name="s",
        num_cores=NUM_CORES, num_subcores=NUM_SUBCORES)

    @functools.partial(
        pl.kernel,
        out_type=jax.ShapeDtypeStruct((n_rows, D_OUT), jnp.float32),
        mesh=mesh,
        compiler_params=pltpu.CompilerParams(needs_layout_passes=False),
        scratch_types=[
            pltpu.VMEM((N_BUF, CHUNK), jnp.int32),
            pltpu.VMEM((N_BUF, CHUNK, D_IN * D_OUT), jnp.float32),
            pltpu.VMEM((N_BUF, CHUNK * D_IN), jnp.float32),
            pltpu.VMEM((N_BUF, CHUNK, D_OUT), jnp.float32),
            pltpu.SemaphoreType.DMA((N_BUF,)),
            pltpu.SemaphoreType.DMA((N_BUF,)),
        ],
    )
    def seg_poly(w_hbm, x_hbm, idx_hbm, out_hbm,
                 idx_v, w_v, x_v, o_v, sem_w, sem_x):
        wid = lax.axis_index("s") * NUM_CORES + lax.axis_index("c")
        base = wid * b_per_w

        def issue(k, b):
            row0 = base + k * CHUNK
            pltpu.sync_copy(idx_hbm.at[pl.ds(row0, CHUNK)], idx_v.at[b])
            pltpu.async_copy(w_hbm.at[idx_v.at[b]], w_v.at[b], sem_w.at[b])
            pltpu.async_copy(x_hbm.at[pl.ds(row0 * D_IN, CHUNK * D_IN)],
                             x_v.at[b], sem_x.at[b])

        def compute(k, b):
            row0 = base + k * CHUNK
            pltpu.make_async_copy(
                w_hbm.at[idx_v.at[b]], w_v.at[b], sem_w.at[b]).wait()
            pltpu.make_async_copy(
                x_hbm.at[pl.ds(row0 * D_IN, CHUNK * D_IN)],
                x_v.at[b], sem_x.at[b]).wait()

            @plsc.parallel_loop(0, CHUNK, unroll=2)
            def row_body(r):
                xv0 = x_v[b, pl.ds(r * D_IN, LANES)]
                xv1 = x_v[b, pl.ds(r * D_IN + LANES, LANES)]
                acc0 = jnp.zeros((LANES,), jnp.float32)
                acc1 = jnp.zeros((LANES,), jnp.float32)
                for i in range(D_IN):
                    xs = xv0[i] if i < LANES else xv1[i - LANES]
                    xb = lax.broadcast(xs, (LANES,))
                    acc0 = acc0 + xb * w_v[b, r, pl.ds(i * D_OUT, LANES)]
                    acc1 = acc1 + xb * w_v[b, r,
                                           pl.ds(i * D_OUT + LANES, LANES)]
                o_v[b, r, pl.ds(0, LANES)] = acc0
                o_v[b, r, pl.ds(LANES, LANES)] = acc1

            pltpu.sync_copy(o_v.at[b], out_hbm.at[pl.ds(row0, CHUNK), :])

        issue(0, 0)

        @pl.loop(0, n_chunks, step=N_BUF)
        def chunk_loop(k0):
            for b in range(N_BUF):
                k = k0 + b

                @pl.when(k + 1 < n_chunks)
                def _():
                    issue(k + 1, (b + 1) % N_BUF)

                compute(k, b)

    return seg_poly


@jax.jit
def kernel(weights, x, weight_indices):
    n_rows = x.shape[0]
    return _make_kernel(n_rows)(weights, x.reshape(-1), weight_indices)
